# baseline (speedup 1.0000x reference)
import functools

import jax
import jax.numpy as jnp
from jax import lax
from jax.experimental import pallas as pl
from jax.experimental.pallas import tpu as pltpu
from jax.experimental.pallas import tpu_sc as plsc

BATCH = 16384
FACTOR = 32
LANES = 16
SLAB = 128
ROWS_PER_SLAB = SLAB // FACTOR
NC, NS = 2, 16
NW = NC * NS
CHUNK = BATCH // NW
NROUND = 4
QCHUNK = CHUNK // NROUND


def _xlane_gather(v, idx):
    return lax.gather(
        v, idx[:, None],
        lax.GatherDimensionNumbers(offset_dims=(), collapsed_slice_dims=(0,),
                                   start_index_map=(0,)),
        (1,), mode=lax.GatherScatterMode.PROMISE_IN_BOUNDS)


def _mf_body(user_hbm, item_hbm, eu_hbm, ei_hbm, ub_hbm, ib_hbm, gb_hbm,
             out_hbm, idx_us, idx_is, idx_uv, idx_iv, sh_u, sh_i, su_buf,
             si_buf, bu_v, bi_v, out_v, gb_v, sems, bsem):
    sid = lax.axis_index("s")
    wid = sid * NC + lax.axis_index("c")
    base = wid * CHUNK

    pltpu.sync_copy(user_hbm.at[pl.ds(base, CHUNK)], idx_uv)
    pltpu.sync_copy(item_hbm.at[pl.ds(base, CHUNK)], idx_iv)
    pltpu.sync_copy(user_hbm.at[pl.ds(base, CHUNK)], sh_u.at[sid])
    pltpu.sync_copy(item_hbm.at[pl.ds(base, CHUNK)], sh_i.at[sid])
    pltpu.sync_copy(sh_u.at[sid], idx_us)
    pltpu.sync_copy(sh_i.at[sid], idx_is)
    pltpu.sync_copy(gb_hbm, gb_v)

    gb = gb_v[pl.ds(0, LANES)]
    lane = lax.iota(jnp.int32, LANES)

    def bias_issue(g, _):
        s = pl.ds(g * LANES, LANES)
        pltpu.async_copy(ub_hbm.at[idx_uv[s]], bu_v.at[s], bsem)
        pltpu.async_copy(ib_hbm.at[idx_iv[s]], bi_v.at[s], bsem)
        return 0

    lax.fori_loop(0, CHUNK // LANES, bias_issue, 0)

    def issue(q, p):
        q0 = q * QCHUNK

        def body(g, _):
            s = pl.ds(q0 + g * LANES, LANES)
            dst = pl.ds(g * LANES, LANES)
            su = lax.shift_right_logical(idx_uv[s], ROWS_PER_SLAB // 2)
            si = lax.shift_right_logical(idx_iv[s], ROWS_PER_SLAB // 2)
            pltpu.async_copy(eu_hbm.at[su], su_buf.at[p].at[dst], sems.at[p])
            pltpu.async_copy(ei_hbm.at[si], si_buf.at[p].at[dst], sems.at[p])
            return 0

        lax.fori_loop(0, QCHUNK // LANES, body, 0)

    def drain(p):
        pltpu.make_async_copy(eu_hbm.at[pl.ds(0, QCHUNK)],
                              su_buf.at[p], sems.at[p]).wait()
        pltpu.make_async_copy(ei_hbm.at[pl.ds(0, QCHUNK)],
                              si_buf.at[p], sems.at[p]).wait()

    def compute(q, p):
        q0 = q * QCHUNK
        su_q = su_buf.at[p]
        si_q = si_buf.at[p]

        def group_body(g, _):
            r0 = g * LANES
            acc = jnp.zeros((LANES,), jnp.float32)
            for t in range(LANES):
                r = r0 + t
                uo = lax.bitwise_and(idx_us[q0 + r], ROWS_PER_SLAB - 1) * FACTOR
                io = lax.bitwise_and(idx_is[q0 + r], ROWS_PER_SLAB - 1) * FACTOR
                prod = (su_q[r, pl.ds(uo, LANES)] * si_q[r, pl.ds(io, LANES)]
                        + su_q[r, pl.ds(uo + LANES, LANES)]
                        * si_q[r, pl.ds(io + LANES, LANES)])
                for k in (8, 4, 2, 1):
                    prod = prod + _xlane_gather(prod, lane ^ k)
                acc = jnp.where(lane == t, prod, acc)
            out_v[pl.ds(q0 + r0, LANES)] = (acc + bu_v[pl.ds(q0 + r0, LANES)]
                                            + bi_v[pl.ds(q0 + r0, LANES)]
                                            + gb)
            return 0

        lax.fori_loop(0, QCHUNK // LANES, group_body, 0)

    issue(0, 0)
    issue(1, 1)
    pltpu.make_async_copy(ub_hbm.at[pl.ds(0, CHUNK)], bu_v, bsem).wait()
    pltpu.make_async_copy(ib_hbm.at[pl.ds(0, CHUNK)], bi_v, bsem).wait()
    for q in range(NROUND):
        drain(q % 2)
        compute(q, q % 2)
        if q + 2 < NROUND:
            issue(q + 2, q % 2)

    pltpu.sync_copy(out_v, out_hbm.at[pl.ds(base, CHUNK)])


@jax.jit
def kernel(user, item, embed_user, embed_item, user_bias, item_bias, bias):
    gb = jnp.broadcast_to(bias.astype(jnp.float32), (LANES,))
    eu_r = embed_user.reshape(-1, SLAB)
    ei_r = embed_item.reshape(-1, SLAB)
    mesh = plsc.VectorSubcoreMesh(core_axis_name="c", subcore_axis_name="s")
    run = pl.kernel(
        _mf_body,
        out_type=jax.ShapeDtypeStruct((BATCH,), jnp.float32),
        mesh=mesh,
        scratch_types=[
            pltpu.SMEM((CHUNK,), jnp.int32),
            pltpu.SMEM((CHUNK,), jnp.int32),
            pltpu.VMEM((CHUNK,), jnp.int32),
            pltpu.VMEM((CHUNK,), jnp.int32),
            pltpu.VMEM_SHARED((NS, CHUNK), jnp.int32),
            pltpu.VMEM_SHARED((NS, CHUNK), jnp.int32),
            pltpu.VMEM((2, QCHUNK, SLAB), jnp.float32),
            pltpu.VMEM((2, QCHUNK, SLAB), jnp.float32),
            pltpu.VMEM((CHUNK,), jnp.float32),
            pltpu.VMEM((CHUNK,), jnp.float32),
            pltpu.VMEM((CHUNK,), jnp.float32),
            pltpu.VMEM((LANES,), jnp.float32),
            pltpu.SemaphoreType.DMA((2,)),
            pltpu.SemaphoreType.DMA,
        ],
    )
    return run(user, item, eu_r, ei_r, user_bias, item_bias, gb)
